# baseline (speedup 1.0000x reference)
import jax
import jax.numpy as jnp
from jax import lax
from jax.experimental import pallas as pl
from jax.experimental.pallas import tpu as pltpu
from jax.experimental.pallas import tpu_sc as plsc

NC, NS, LANES = 2, 16, 16
NW = NC * NS
EMBED = 32
OBLK = 512
OCTS = 8
HBAND = 200 // (NW // OCTS)
SW = 56
GATHER = 64
SLICES = OBLK // GATHER


def _sc_body(ids_hbm, ctxi_hbm, base_hbm, sup_hbm, out_hbm,
             stage, idt, brows, srows, obuf, sem_g, sem_o):
    wid = lax.axis_index("s") * NC + lax.axis_index("c")
    oct_i = lax.rem(wid, OCTS)
    band = wid // OCTS
    b0 = oct_i * OBLK
    h0 = band * HBAND
    s0 = 8 * (h0 // 8)
    off = h0 - s0
    iota = lax.iota(jnp.int32, LANES)

    pltpu.sync_copy(ids_hbm.at[pl.ds(b0, OBLK), pl.ds(s0, SW)], stage)

    def t_ids(hl, c):
        hv = jnp.full((LANES,), hl + off, jnp.int32)
        for tb in range(OBLK // LANES):
            g = plsc.load_gather(stage, [iota + tb * LANES, hv])
            idt[hl, pl.ds(tb * LANES, LANES)] = g
        return c

    lax.fori_loop(0, HBAND, t_ids, 0)

    pltpu.sync_copy(ctxi_hbm.at[pl.ds(b0, OBLK), pl.ds(s0, SW)], stage)

    for si in range(SLICES - 1):
        idx = idt.at[0, pl.ds(si * GATHER, GATHER)]
        pltpu.async_copy(base_hbm.at[idx], brows.at[si], sem_g)
        pltpu.async_copy(sup_hbm.at[idx], srows.at[si], sem_g)

    row_even = iota * 2
    row_odd = iota * 2 + 1
    himask = jnp.full((LANES,), -65536, jnp.int32)

    def main(hl, c):
        po = lax.rem(hl, 2)
        hv = jnp.full((LANES,), hl + off, jnp.int32)

        @pl.when(hl >= 2)
        def _():
            pltpu.make_async_copy(
                obuf.at[po], out_hbm.at[0, :, pl.ds(b0, OBLK)], sem_o).wait()

        for si in range(SLICES):
            hf = hl + (si + SLICES - 1) // SLICES
            sif = (si + SLICES - 1) % SLICES

            @pl.when(hf < HBAND)
            def _(hf=hf, sif=sif):
                idx = idt.at[hf, pl.ds(sif * GATHER, GATHER)]
                pltpu.async_copy(base_hbm.at[idx], brows.at[sif], sem_g)
                pltpu.async_copy(sup_hbm.at[idx], srows.at[sif], sem_g)

            idx_c = idt.at[hl, pl.ds(si * GATHER, GATHER)]
            pltpu.make_async_copy(base_hbm.at[idx_c], brows.at[si], sem_g).wait()
            pltpu.make_async_copy(sup_hbm.at[idx_c], srows.at[si], sem_g).wait()

            for tb in range(GATHER // LANES):
                t0g = si * GATHER + tb * LANES
                cvi = plsc.load_gather(stage, [iota + t0g, hv])
                cv = plsc.bitcast(cvi, jnp.float32)
                for j in range(0, LANES, 4):
                    ts = [tb * LANES + j + k for k in range(4)]
                    cbs = [jnp.full((LANES,), cv[j + k]) for k in range(4)]
                    bis = [plsc.bitcast(brows[si, t, :], jnp.int32) for t in ts]
                    sis = [plsc.bitcast(srows[si, t, :], jnp.int32) for t in ts]
                    bevens = [plsc.bitcast(x << 16, jnp.float32) for x in bis]
                    bodds = [plsc.bitcast(x & himask, jnp.float32) for x in bis]
                    sevens = [plsc.bitcast(x << 16, jnp.float32) for x in sis]
                    sodds = [plsc.bitcast(x & himask, jnp.float32) for x in sis]
                    tvs = [jnp.full((LANES,), t0g + j + k, jnp.int32)
                           for k in range(4)]
                    for k in range(4):
                        ve = bevens[k] + cbs[k] * sevens[k]
                        vo = bodds[k] + cbs[k] * sodds[k]
                        plsc.store_scatter(obuf.at[po], [row_even, tvs[k]], ve)
                        plsc.store_scatter(obuf.at[po], [row_odd, tvs[k]], vo)

        pltpu.async_copy(obuf.at[po], out_hbm.at[h0 + hl, :, pl.ds(b0, OBLK)], sem_o)
        return c

    lax.fori_loop(0, HBAND, main, 0)

    pltpu.make_async_copy(obuf.at[0], out_hbm.at[0, :, pl.ds(b0, OBLK)], sem_o).wait()
    pltpu.make_async_copy(obuf.at[1], out_hbm.at[0, :, pl.ds(b0, OBLK)], sem_o).wait()


def kernel(input_ids, context_vector, base_table, superposed_table):
    b, h = input_ids.shape
    ids = input_ids.astype(jnp.int32)
    ctxi = lax.bitcast_convert_type(context_vector, jnp.int32)
    base_bf = base_table.astype(jnp.bfloat16)
    sup_bf = superposed_table.astype(jnp.bfloat16)
    mesh = plsc.VectorSubcoreMesh(core_axis_name="c", subcore_axis_name="s",
                                  num_cores=NC, num_subcores=NS)
    out_t = pl.kernel(
        _sc_body,
        out_type=jax.ShapeDtypeStruct((h, EMBED, b), jnp.float32),
        mesh=mesh,
        scratch_types=[
            pltpu.VMEM((OBLK, SW), jnp.int32),
            pltpu.VMEM((HBAND, OBLK), jnp.int32),
            pltpu.VMEM((SLICES, GATHER, EMBED), jnp.bfloat16),
            pltpu.VMEM((SLICES, GATHER, EMBED), jnp.bfloat16),
            pltpu.VMEM((2, EMBED, OBLK), jnp.float32),
            pltpu.SemaphoreType.DMA,
            pltpu.SemaphoreType.DMA,
        ],
        compiler_params=pltpu.CompilerParams(
            use_tc_tiling_on_sc=False, needs_layout_passes=False),
    )(ids, ctxi, base_bf, sup_bf)
    return out_t.transpose(2, 0, 1)


# speedup vs baseline: 1.1157x; 1.1157x over previous
import jax
import jax.numpy as jnp
from jax import lax
from jax.experimental import pallas as pl
from jax.experimental.pallas import tpu as pltpu
from jax.experimental.pallas import tpu_sc as plsc

NC, NS, LANES = 2, 16, 16
NW = NC * NS
EMBED = 32
OBLK = 512
OCTS = 8
HBAND = 200 // (NW // OCTS)
SW = 56
GATHER = 64
SLICES = OBLK // GATHER


def _sc_body(ids_hbm, ctxi_hbm, base_hbm, sup_hbm, out_hbm,
             stage, idt, brows, srows, obuf, sem_g, sem_o):
    wid = lax.axis_index("s") * NC + lax.axis_index("c")
    oct_i = lax.rem(wid, OCTS)
    band = wid // OCTS
    b0 = oct_i * OBLK
    h0 = band * HBAND
    s0 = 8 * (h0 // 8)
    off = h0 - s0
    iota = lax.iota(jnp.int32, LANES)

    pltpu.sync_copy(ids_hbm.at[pl.ds(b0, OBLK), pl.ds(s0, SW)], stage)

    def t_ids(hl, c):
        hv = jnp.full((LANES,), hl + off, jnp.int32)
        for tb in range(OBLK // LANES):
            g = plsc.load_gather(stage, [iota + tb * LANES, hv])
            idt[hl, pl.ds(tb * LANES, LANES)] = g
        return c

    lax.fori_loop(0, HBAND, t_ids, 0)

    pltpu.sync_copy(ctxi_hbm.at[pl.ds(b0, OBLK), pl.ds(s0, SW)], stage)

    for si in range(SLICES - 1):
        idx = idt.at[0, pl.ds(si * GATHER, GATHER)]
        pltpu.async_copy(base_hbm.at[idx], brows.at[si], sem_g)
        pltpu.async_copy(sup_hbm.at[idx], srows.at[si], sem_g)

    ds0 = pl.ds(0, LANES)
    ds1 = pl.ds(LANES, LANES)
    row0 = iota
    row1 = iota + LANES

    def main(hl, c):
        po = lax.rem(hl, 2)
        hv = jnp.full((LANES,), hl + off, jnp.int32)

        @pl.when(hl >= 2)
        def _():
            pltpu.make_async_copy(
                obuf.at[po], out_hbm.at[0, :, pl.ds(b0, OBLK)], sem_o).wait()

        for si in range(SLICES):
            hf = hl + (si + SLICES - 1) // SLICES
            sif = (si + SLICES - 1) % SLICES

            @pl.when(hf < HBAND)
            def _(hf=hf, sif=sif):
                idx = idt.at[hf, pl.ds(sif * GATHER, GATHER)]
                pltpu.async_copy(base_hbm.at[idx], brows.at[sif], sem_g)
                pltpu.async_copy(sup_hbm.at[idx], srows.at[sif], sem_g)

            idx_c = idt.at[hl, pl.ds(si * GATHER, GATHER)]
            pltpu.make_async_copy(base_hbm.at[idx_c], brows.at[si], sem_g).wait()
            pltpu.make_async_copy(sup_hbm.at[idx_c], srows.at[si], sem_g).wait()

            for tb in range(GATHER // LANES):
                t0g = si * GATHER + tb * LANES
                cvi = plsc.load_gather(stage, [iota + t0g, hv])
                cv = plsc.bitcast(cvi, jnp.float32)
                for j in range(0, LANES, 4):
                    ts = [tb * LANES + j + k for k in range(4)]
                    cbs = [jnp.full((LANES,), cv[j + k]) for k in range(4)]
                    bs = [(brows[si, t, ds0], brows[si, t, ds1]) for t in ts]
                    ss = [(srows[si, t, ds0], srows[si, t, ds1]) for t in ts]
                    vs = [(b0_ + cb * s0_, b1_ + cb * s1_)
                          for (b0_, b1_), (s0_, s1_), cb in zip(bs, ss, cbs)]
                    tvs = [jnp.full((LANES,), t0g + j + k, jnp.int32)
                           for k in range(4)]
                    for (v0_, v1_), tv in zip(vs, tvs):
                        plsc.store_scatter(obuf.at[po], [row0, tv], v0_)
                        plsc.store_scatter(obuf.at[po], [row1, tv], v1_)

        pltpu.async_copy(obuf.at[po], out_hbm.at[h0 + hl, :, pl.ds(b0, OBLK)], sem_o)
        return c

    lax.fori_loop(0, HBAND, main, 0)

    pltpu.make_async_copy(obuf.at[0], out_hbm.at[0, :, pl.ds(b0, OBLK)], sem_o).wait()
    pltpu.make_async_copy(obuf.at[1], out_hbm.at[0, :, pl.ds(b0, OBLK)], sem_o).wait()


def kernel(input_ids, context_vector, base_table, superposed_table):
    b, h = input_ids.shape
    ids = input_ids.astype(jnp.int32)
    ctxi = lax.bitcast_convert_type(context_vector, jnp.int32)
    mesh = plsc.VectorSubcoreMesh(core_axis_name="c", subcore_axis_name="s",
                                  num_cores=NC, num_subcores=NS)
    out_t = pl.kernel(
        _sc_body,
        out_type=jax.ShapeDtypeStruct((h, EMBED, b), jnp.float32),
        mesh=mesh,
        scratch_types=[
            pltpu.VMEM((OBLK, SW), jnp.int32),
            pltpu.VMEM((HBAND, OBLK), jnp.int32),
            pltpu.VMEM((SLICES, GATHER, EMBED), jnp.float32),
            pltpu.VMEM((SLICES, GATHER, EMBED), jnp.float32),
            pltpu.VMEM((2, EMBED, OBLK), jnp.float32),
            pltpu.SemaphoreType.DMA,
            pltpu.SemaphoreType.DMA,
        ],
        compiler_params=pltpu.CompilerParams(
            use_tc_tiling_on_sc=False, needs_layout_passes=False),
    )(ids, ctxi, base_table, superposed_table)
    return out_t.transpose(2, 0, 1)
